# baseline (speedup 1.0000x reference)
import functools

import jax
import jax.numpy as jnp
from jax import lax
from jax.experimental import pallas as pl
from jax.experimental.pallas import tpu as pltpu
from jax.experimental.pallas import tpu_sc as plsc

N = 10000
NE = 80000
D = 256
R = 64
G = 128
NEP = 81920
EPT = NEP // 16
NCHUNK = 80
CH = 64
REC = 3 * CH
HALF = 5000
HP = 5008
RPT = HP // 16
NPR = 640
TRASH_DST = 10008


def _mesh():
    return plsc.VectorSubcoreMesh(core_axis_name="c", subcore_axis_name="s")


_SC_PARAMS = pltpu.CompilerParams(use_tc_tiling_on_sc=False,
                                  needs_layout_passes=False)


def _rsqrt_f32(v):
    i = plsc.bitcast(v, jnp.int32)
    y = plsc.bitcast(jnp.int32(0x5F3759DF) - (i >> 1), jnp.float32)
    for _ in range(3):
        y = y * (1.5 - 0.5 * v * y * y)
    return y


def _k0_body(dsts_ref, zeros_ref, iota_ref, dinv_ref, degv, dstb, idxv, tbuf,
             shdeg):
    cid = lax.axis_index("c")
    sid = lax.axis_index("s")
    pltpu.sync_copy(zeros_ref, degv)
    pltpu.sync_copy(iota_ref, idxv)
    for h in range(2):
        pltpu.sync_copy(zeros_ref.at[h].at[pl.ds(sid * 40, 40)],
                        shdeg.at[h].at[pl.ds(sid * 40, 40)])
    plsc.subcore_barrier()
    for h in range(2):
        pltpu.sync_copy(dsts_ref.at[h].at[pl.ds(sid * EPT, EPT)], dstb)

        def body(j, carry, h=h):
            idx16 = dstb[pl.ds(j * 16, 16)]
            cnt, last = plsc.scan_count(idx16)
            plsc.addupdate_scatter(degv.at[h], [idx16 >> 4, idx16 & 15],
                                   cnt.astype(jnp.float32), mask=last)
            return carry

        lax.fori_loop(0, EPT // 16, body, 0)
    for h in range(2):
        pltpu.sync_copy(degv.at[h], shdeg.at[h].at[idxv], add=True)
    plsc.subcore_barrier()
    pltpu.sync_copy(shdeg.at[cid].at[pl.ds(sid * 40, 40)], tbuf)
    for k in range(40):
        d = tbuf[k]
        y = _rsqrt_f32(jnp.maximum(d, 1.0))
        tbuf[k] = jnp.where(d > 0.0, y, 0.0)
    pltpu.sync_copy(tbuf, dinv_ref.at[cid].at[pl.ds(sid * 40, 40)])


def _k0(dsts, zeros_nt, iota_npr):
    return pl.kernel(
        _k0_body,
        out_type=jax.ShapeDtypeStruct((2, NPR, 16), jnp.float32),
        mesh=_mesh(),
        compiler_params=_SC_PARAMS,
        scratch_types=[
            pltpu.VMEM((2, NPR, 16), jnp.float32),
            pltpu.VMEM((EPT,), jnp.int32),
            pltpu.VMEM((NPR,), jnp.int32),
            pltpu.VMEM((40, 16), jnp.float32),
            pltpu.VMEM_SHARED((2, NPR, 16), jnp.float32),
        ],
    )(dsts, zeros_nt, iota_npr)


def _k2_body(xs_ref, rel_ref, rec_ref, dfl_ref, zrow_ref, agg_ref, relv,
             recb0, lidb0, lidb1, xrow0, xrow1, dvb, sem0, sem1, ssem0,
             ssem1, aggsh):
    cid = lax.axis_index("c")
    sid = lax.axis_index("s")
    lidb = (lidb0, lidb1)
    xrow = (xrow0, xrow1)
    sem = (sem0, sem1)
    ssem = (ssem0, ssem1)
    pltpu.sync_copy(rel_ref, relv)
    pltpu.sync_copy(zrow_ref, xrow0)
    r0 = pl.multiple_of(((sid * RPT) >> 3) << 3, 8)
    for k in range(0, 320, CH):
        pltpu.sync_copy(xrow0, aggsh.at[pl.ds(r0 + k, CH)])
    plsc.subcore_barrier()
    lane = lax.iota(jnp.int32, 16)
    base = cid * HALF

    def pair(j, carry):
        descs = []
        roff = pl.multiple_of((sid * NCHUNK + 2 * j) * REC, 8)
        pltpu.sync_copy(rec_ref.at[pl.ds(roff, 2 * REC)],
                        recb0.at[pl.ds(0, 2 * REC)])
        for b in range(2):
            o = b * REC
            descs.append(
                pltpu.async_copy(xs_ref.at[recb0.at[pl.ds(o, CH)]], xrow[b],
                                 sem[b]))
            for g in range(CH // 16):
                d16 = recb0[pl.ds(o + CH + g * 16, 16)]
                lid = d16 - base
                own = (lid >= 0) & (lid < HALF)
                lidb[b][pl.ds(g * 16, 16)] = jnp.where(
                    own, lid, HALF + (lane & 7))
        sdescs = []
        for b in range(2):
            descs[b].wait()

            def edge(e, c2, b=b):
                d_e = recb0[pl.ds(b * REC + CH + e, 16)][0]

                @pl.when((d_e >= base) & (d_e < base + HALF))
                def _():
                    et_e = recb0[pl.ds(b * REC + 2 * CH + e, 16)][0]
                    for jj in range(D // 16):
                        sl = pl.ds(jj * 16, 16)
                        xrow[b][e, sl] = xrow[b][e, sl] * relv[et_e, sl]

                return c2

            lax.fori_loop(0, CH, edge, 0)
            sdescs.append(
                pltpu.async_copy(xrow[b], aggsh.at[lidb[b]], ssem[b],
                                 add=True))
        for b in range(2):
            sdescs[b].wait()
        return carry

    lax.fori_loop(0, NCHUNK // 2, pair, 0)
    plsc.subcore_barrier()
    pltpu.sync_copy(dfl_ref.at[pl.ds(cid * HALF + r0, 320)],
                    dvb.at[pl.ds(0, 320)])
    for k in range(0, 320, CH):
        pltpu.sync_copy(aggsh.at[pl.ds(r0 + k, CH)], xrow0)

        def scale(rr, c3, k=k):
            dv = dvb[pl.ds(k + rr, 16)][0]
            for jj in range(D // 16):
                sl = pl.ds(jj * 16, 16)
                xrow0[rr, sl] = xrow0[rr, sl] * dv
            return c3

        lax.fori_loop(0, CH, scale, 0)
        pltpu.sync_copy(xrow0, agg_ref.at[cid].at[pl.ds(r0 + k, CH)])


def _k2(xs, rel_lo, rec_h, dfl_h, zrow):
    return pl.kernel(
        _k2_body,
        out_type=jax.ShapeDtypeStruct((2, HP, D), jnp.float32),
        mesh=_mesh(),
        compiler_params=_SC_PARAMS,
        scratch_types=[
            pltpu.VMEM((R, D), jnp.float32),
            pltpu.VMEM((2 * REC + 16,), jnp.int32),
            pltpu.VMEM((CH,), jnp.int32),
            pltpu.VMEM((CH,), jnp.int32),
            pltpu.VMEM((CH, D), jnp.float32),
            pltpu.VMEM((CH, D), jnp.float32),
            pltpu.VMEM((336,), jnp.float32),
            pltpu.SemaphoreType.DMA,
            pltpu.SemaphoreType.DMA,
            pltpu.SemaphoreType.DMA,
            pltpu.SemaphoreType.DMA,
            pltpu.VMEM_SHARED((HP, D), jnp.float32),
        ],
    )(xs, rel_lo, rec_h, dfl_h, zrow)


def _k3_body(ain_ref, aout_ref, xh_ref, rel_ref, lr2_ref, wi_ref, wo_ref,
             wl_ref, wr_ref, b_ref, g_ref, be_ref, h_ref, relw_ref, *, relu):
    out = jnp.dot(ain_ref[...], wi_ref[...], preferred_element_type=jnp.float32)
    out = out + jnp.dot(aout_ref[...], wo_ref[...],
                        preferred_element_type=jnp.float32)
    out = out + jnp.dot(xh_ref[...], lr2_ref[...] * wl_ref[...],
                        preferred_element_type=jnp.float32)
    out = out * (1.0 / 3.0) + b_ref[...]
    mean = jnp.mean(out, axis=0, keepdims=True)
    cent = out - mean
    var = jnp.mean(cent * cent, axis=0, keepdims=True)
    out = cent * lax.rsqrt(var + 1e-5) * g_ref[...] + be_ref[...]
    out = jnp.tanh(out)
    if relu:
        out = jnp.maximum(out, 0.0)
    h_ref[...] = out
    relw_ref[...] = jnp.dot(rel_ref[...], wr_ref[...],
                            preferred_element_type=jnp.float32)


def _k3(ain, aout, xh, rel_all, p, relu):
    return pl.pallas_call(
        functools.partial(_k3_body, relu=relu),
        out_shape=[
            jax.ShapeDtypeStruct((N, D), jnp.float32),
            jax.ShapeDtypeStruct((R + 1, D), jnp.float32),
        ],
    )(ain, aout, xh, rel_all, rel_all[R].reshape(D, 1), p['w_in'], p['w_out'],
      p['w_loop'], p['w_rel'], p['bias'].reshape(1, D),
      p['bn_gamma'].reshape(1, D), p['bn_beta'].reshape(1, D))


def _p0_body(x_ref, di_ref, do_ref, xi_ref, xo_ref):
    x = x_ref[...]
    xi_ref[...] = x * di_ref[...]
    xo_ref[...] = x * do_ref[...]


def _p0(x, di, do):
    return pl.pallas_call(
        _p0_body,
        out_shape=[
            jax.ShapeDtypeStruct((N, D), jnp.float32),
            jax.ShapeDtypeStruct((N, D), jnp.float32),
        ],
    )(x, di, do)


def _k4_body(h_ref, batch_ref, rlab_ref, rel_emb_ref, lw_ref, lb_ref, o_ref):
    onehot = (lax.broadcasted_iota(jnp.int32, (G, N), 0)
              == batch_ref[...]).astype(jnp.float32)
    seg = jnp.dot(onehot, h_ref[...], preferred_element_type=jnp.float32)
    cnt = jnp.sum(onehot, axis=1, keepdims=True)
    pooled = seg / jnp.maximum(cnt, 1.0)
    onehot_r = (lax.broadcasted_iota(jnp.int32, (G, R), 1)
                == rlab_ref[...]).astype(jnp.float32)
    rel_sel = jnp.dot(onehot_r, rel_emb_ref[...],
                      preferred_element_type=jnp.float32)
    z = jnp.concatenate([pooled, rel_sel], axis=1)
    o_ref[...] = jnp.dot(z, lw_ref[...],
                         preferred_element_type=jnp.float32) + lb_ref[...]


def _k4(h, batch2d, rlab2d, rel_emb, lwp, lbp):
    return pl.pallas_call(
        _k4_body,
        out_shape=jax.ShapeDtypeStruct((G, 128), jnp.float32),
    )(h, batch2d, rlab2d, rel_emb, lwp, lbp)


def kernel(x, params, edge_index, edge_type, batch, rel_labels, drop_prob):
    i32 = jnp.int32
    f32 = jnp.float32
    npad = NEP - NE
    pad0 = jnp.zeros((npad,), i32)
    padt = jnp.full((npad,), TRASH_DST, i32)
    src = edge_index[1].astype(i32)
    dst = edge_index[0].astype(i32)
    et = edge_type.astype(i32)
    srcs = jnp.stack([jnp.concatenate([src[:NE], pad0]),
                      jnp.concatenate([src[NE:], pad0])])
    dsts = jnp.stack([jnp.concatenate([dst[:NE], padt]),
                      jnp.concatenate([dst[NE:], padt])])
    ets = jnp.stack([jnp.concatenate([et[:NE], pad0]),
                     jnp.concatenate([et[NE:], pad0])])
    rec = jnp.stack([s.reshape(2, 16, NCHUNK, CH) for s in (srcs, dsts, ets)],
                    axis=3).reshape(2, 16 * NCHUNK * REC)
    zeros_nt = jnp.zeros((2, NPR, 16), f32)
    iota_npr = jnp.arange(NPR, dtype=i32)
    zrow = jnp.zeros((CH, D), f32)

    dinv = _k0(dsts, zeros_nt, iota_npr)
    dflat_full = dinv.reshape(2, NPR * 16)
    dflat = dflat_full[:, :N]
    di = dflat[0].reshape(N, 1)
    do = dflat[1].reshape(N, 1)

    rel_all = jnp.concatenate(
        [params['rel_graph_emb'], params['conv1']['loop_rel']], axis=0)
    h = x
    hsi, hso = _p0(x, di, do)
    names = ['conv1', 'conv2', 'conv3']
    for li, name in enumerate(names):
        p = params[name]
        rel_lo = rel_all[:R]
        a_in = _k2(hsi, rel_lo, rec[0], dflat_full[0], zrow)
        a_out = _k2(hso, rel_lo, rec[1], dflat_full[1], zrow)
        a_in = jnp.concatenate([a_in[0, :HALF], a_in[1, :HALF]], axis=0)
        a_out = jnp.concatenate([a_out[0, :HALF], a_out[1, :HALF]], axis=0)
        h, relw = _k3(a_in, a_out, h, rel_all, p, relu=(li < 2))
        if li < 2:
            hsi, hso = _p0(h, di, do)
            rel_all = jnp.concatenate(
                [relw[:R], params[names[li + 1]]['loop_rel']], axis=0)

    lwp = jnp.pad(params['lin_w'], ((0, 0), (0, 126)))
    lbp = jnp.pad(params['lin_b'].reshape(1, 2), ((0, 0), (0, 126)))
    out_full = _k4(h, batch.reshape(1, N).astype(i32),
                   rel_labels.reshape(G, 1).astype(i32),
                   params['rel_emb'], lwp, lbp)
    return out_full[:, :2]


# speedup vs baseline: 1.0281x; 1.0281x over previous
import functools

import jax
import jax.numpy as jnp
from jax import lax
from jax.experimental import pallas as pl
from jax.experimental.pallas import tpu as pltpu
from jax.experimental.pallas import tpu_sc as plsc

N = 10000
NE = 80000
D = 256
R = 64
G = 128
NEP = 81920
EPT = NEP // 16
NCHUNK = 80
CH = 64
REC = 3 * CH
HALF = 5000
HP = 5008
RPT = HP // 16
NPR = 640
TRASH_DST = 10008


def _mesh():
    return plsc.VectorSubcoreMesh(core_axis_name="c", subcore_axis_name="s")


_SC_PARAMS = pltpu.CompilerParams(use_tc_tiling_on_sc=False,
                                  needs_layout_passes=False)


def _rsqrt_f32(v):
    i = plsc.bitcast(v, jnp.int32)
    y = plsc.bitcast(jnp.int32(0x5F3759DF) - (i >> 1), jnp.float32)
    for _ in range(3):
        y = y * (1.5 - 0.5 * v * y * y)
    return y


def _k0_body(dsts_ref, zeros_ref, iota_ref, dinv_ref, degv, dstb, idxv, tbuf,
             shdeg):
    cid = lax.axis_index("c")
    sid = lax.axis_index("s")
    pltpu.sync_copy(zeros_ref, degv)
    pltpu.sync_copy(iota_ref, idxv)
    for h in range(2):
        pltpu.sync_copy(zeros_ref.at[h].at[pl.ds(sid * 40, 40)],
                        shdeg.at[h].at[pl.ds(sid * 40, 40)])
    plsc.subcore_barrier()
    for h in range(2):
        pltpu.sync_copy(dsts_ref.at[h].at[pl.ds(sid * EPT, EPT)], dstb)

        def body(j, carry, h=h):
            idx16 = dstb[pl.ds(j * 16, 16)]
            cnt, last = plsc.scan_count(idx16)
            plsc.addupdate_scatter(degv.at[h], [idx16 >> 4, idx16 & 15],
                                   cnt.astype(jnp.float32), mask=last)
            return carry

        lax.fori_loop(0, EPT // 16, body, 0)
    for h in range(2):
        pltpu.sync_copy(degv.at[h], shdeg.at[h].at[idxv], add=True)
    plsc.subcore_barrier()
    pltpu.sync_copy(shdeg.at[cid].at[pl.ds(sid * 40, 40)], tbuf)
    for k in range(40):
        d = tbuf[k]
        y = _rsqrt_f32(jnp.maximum(d, 1.0))
        tbuf[k] = jnp.where(d > 0.0, y, 0.0)
    pltpu.sync_copy(tbuf, dinv_ref.at[cid].at[pl.ds(sid * 40, 40)])


def _k0(dsts, zeros_nt, iota_npr):
    return pl.kernel(
        _k0_body,
        out_type=jax.ShapeDtypeStruct((2, NPR, 16), jnp.float32),
        mesh=_mesh(),
        compiler_params=_SC_PARAMS,
        scratch_types=[
            pltpu.VMEM((2, NPR, 16), jnp.float32),
            pltpu.VMEM((EPT,), jnp.int32),
            pltpu.VMEM((NPR,), jnp.int32),
            pltpu.VMEM((40, 16), jnp.float32),
            pltpu.VMEM_SHARED((2, NPR, 16), jnp.float32),
        ],
    )(dsts, zeros_nt, iota_npr)


def _k2_body(xs_ref, rel_ref, rec_ref, dfl_ref, zrow_ref, agg_ref, relv,
             recb0, recb1, srcb0, srcb1, lidb0, lidb1, xrow0, xrow1, dvb,
             sem0, sem1, ssem0, ssem1, aggsh):
    cid = lax.axis_index("c")
    sid = lax.axis_index("s")
    recb = (recb0, recb1)
    srcb = (srcb0, srcb1)
    lidb = (lidb0, lidb1)
    xrow = (xrow0, xrow1)
    sem = (sem0, sem1)
    ssem = (ssem0, ssem1)
    pltpu.sync_copy(rel_ref, relv)
    pltpu.sync_copy(zrow_ref, xrow0)
    r0 = pl.multiple_of(((sid * RPT) >> 3) << 3, 8)
    for k in range(0, 320, CH):
        pltpu.sync_copy(xrow0, aggsh.at[pl.ds(r0 + k, CH)])
    plsc.subcore_barrier()
    lane = lax.iota(jnp.int32, 16)
    base = cid * HALF

    def pair(j, carry):
        descs = []
        for b in range(2):
            i = 2 * j + b
            roff = pl.multiple_of((sid * NCHUNK + i) * REC, 8)
            pltpu.sync_copy(rec_ref.at[pl.ds(roff, REC)],
                            recb[b].at[pl.ds(0, REC)])
            for g in range(CH // 16):
                s16 = recb[b][pl.ds(g * 16, 16)]
                d16 = recb[b][pl.ds(CH + g * 16, 16)]
                srcb[b][pl.ds(g * 16, 16)] = s16
                lid = d16 - base
                own = (lid >= 0) & (lid < HALF)
                lidb[b][pl.ds(g * 16, 16)] = jnp.where(
                    own, lid, HALF + (lane & 7))
            descs.append(pltpu.async_copy(xs_ref.at[srcb[b]], xrow[b], sem[b]))
        sdescs = []
        for b in range(2):
            descs[b].wait()

            def edge(e, c2, b=b):
                d_e = recb[b][pl.ds(CH + e, 16)][0]

                @pl.when((d_e >= base) & (d_e < base + HALF))
                def _():
                    et_e = recb[b][pl.ds(2 * CH + e, 16)][0]
                    for jj in range(D // 16):
                        sl = pl.ds(jj * 16, 16)
                        xrow[b][e, sl] = xrow[b][e, sl] * relv[et_e, sl]

                return c2

            lax.fori_loop(0, CH, edge, 0)
            sdescs.append(
                pltpu.async_copy(xrow[b], aggsh.at[lidb[b]], ssem[b],
                                 add=True))
        for b in range(2):
            sdescs[b].wait()
        return carry

    lax.fori_loop(0, NCHUNK // 2, pair, 0)
    plsc.subcore_barrier()
    pltpu.sync_copy(dfl_ref.at[pl.ds(cid * HALF + r0, 320)],
                    dvb.at[pl.ds(0, 320)])
    for k in range(0, 320, CH):
        pltpu.sync_copy(aggsh.at[pl.ds(r0 + k, CH)], xrow0)

        def scale(rr, c3, k=k):
            dv = dvb[pl.ds(k + rr, 16)][0]
            for jj in range(D // 16):
                sl = pl.ds(jj * 16, 16)
                xrow0[rr, sl] = xrow0[rr, sl] * dv
            return c3

        lax.fori_loop(0, CH, scale, 0)
        pltpu.sync_copy(xrow0, agg_ref.at[cid].at[pl.ds(r0 + k, CH)])


def _k2(xs, rel_lo, rec_h, dfl_h, zrow):
    return pl.kernel(
        _k2_body,
        out_type=jax.ShapeDtypeStruct((2, HP, D), jnp.float32),
        mesh=_mesh(),
        compiler_params=_SC_PARAMS,
        scratch_types=[
            pltpu.VMEM((R, D), jnp.float32),
            pltpu.VMEM((REC + 16,), jnp.int32),
            pltpu.VMEM((REC + 16,), jnp.int32),
            pltpu.VMEM((CH,), jnp.int32),
            pltpu.VMEM((CH,), jnp.int32),
            pltpu.VMEM((CH,), jnp.int32),
            pltpu.VMEM((CH,), jnp.int32),
            pltpu.VMEM((CH, D), jnp.float32),
            pltpu.VMEM((CH, D), jnp.float32),
            pltpu.VMEM((336,), jnp.float32),
            pltpu.SemaphoreType.DMA,
            pltpu.SemaphoreType.DMA,
            pltpu.SemaphoreType.DMA,
            pltpu.SemaphoreType.DMA,
            pltpu.VMEM_SHARED((HP, D), jnp.float32),
        ],
    )(xs, rel_lo, rec_h, dfl_h, zrow)


def _k3_body(ain_ref, aout_ref, xh_ref, rel_ref, lr2_ref, wi_ref, wo_ref,
             wl_ref, wr_ref, b_ref, g_ref, be_ref, h_ref, relw_ref, *, relu):
    out = jnp.dot(ain_ref[...], wi_ref[...], preferred_element_type=jnp.float32)
    out = out + jnp.dot(aout_ref[...], wo_ref[...],
                        preferred_element_type=jnp.float32)
    out = out + jnp.dot(xh_ref[...], lr2_ref[...] * wl_ref[...],
                        preferred_element_type=jnp.float32)
    out = out * (1.0 / 3.0) + b_ref[...]
    mean = jnp.mean(out, axis=0, keepdims=True)
    cent = out - mean
    var = jnp.mean(cent * cent, axis=0, keepdims=True)
    out = cent * lax.rsqrt(var + 1e-5) * g_ref[...] + be_ref[...]
    out = jnp.tanh(out)
    if relu:
        out = jnp.maximum(out, 0.0)
    h_ref[...] = out
    relw_ref[...] = jnp.dot(rel_ref[...], wr_ref[...],
                            preferred_element_type=jnp.float32)


def _k3(ain, aout, xh, rel_all, p, relu):
    return pl.pallas_call(
        functools.partial(_k3_body, relu=relu),
        out_shape=[
            jax.ShapeDtypeStruct((N, D), jnp.float32),
            jax.ShapeDtypeStruct((R + 1, D), jnp.float32),
        ],
    )(ain, aout, xh, rel_all, rel_all[R].reshape(D, 1), p['w_in'], p['w_out'],
      p['w_loop'], p['w_rel'], p['bias'].reshape(1, D),
      p['bn_gamma'].reshape(1, D), p['bn_beta'].reshape(1, D))


def _p0_body(x_ref, di_ref, do_ref, xi_ref, xo_ref):
    x = x_ref[...]
    xi_ref[...] = x * di_ref[...]
    xo_ref[...] = x * do_ref[...]


def _p0(x, di, do):
    return pl.pallas_call(
        _p0_body,
        out_shape=[
            jax.ShapeDtypeStruct((N, D), jnp.float32),
            jax.ShapeDtypeStruct((N, D), jnp.float32),
        ],
    )(x, di, do)


def _k4_body(h_ref, batch_ref, rlab_ref, rel_emb_ref, lw_ref, lb_ref, o_ref):
    onehot = (lax.broadcasted_iota(jnp.int32, (G, N), 0)
              == batch_ref[...]).astype(jnp.float32)
    seg = jnp.dot(onehot, h_ref[...], preferred_element_type=jnp.float32)
    cnt = jnp.sum(onehot, axis=1, keepdims=True)
    pooled = seg / jnp.maximum(cnt, 1.0)
    onehot_r = (lax.broadcasted_iota(jnp.int32, (G, R), 1)
                == rlab_ref[...]).astype(jnp.float32)
    rel_sel = jnp.dot(onehot_r, rel_emb_ref[...],
                      preferred_element_type=jnp.float32)
    z = jnp.concatenate([pooled, rel_sel], axis=1)
    o_ref[...] = jnp.dot(z, lw_ref[...],
                         preferred_element_type=jnp.float32) + lb_ref[...]


def _k4(h, batch2d, rlab2d, rel_emb, lwp, lbp):
    return pl.pallas_call(
        _k4_body,
        out_shape=jax.ShapeDtypeStruct((G, 128), jnp.float32),
    )(h, batch2d, rlab2d, rel_emb, lwp, lbp)


def kernel(x, params, edge_index, edge_type, batch, rel_labels, drop_prob):
    i32 = jnp.int32
    f32 = jnp.float32
    npad = NEP - NE
    pad0 = jnp.zeros((npad,), i32)
    padt = jnp.full((npad,), TRASH_DST, i32)
    src = edge_index[1].astype(i32)
    dst = edge_index[0].astype(i32)
    et = edge_type.astype(i32)
    srcs = jnp.stack([jnp.concatenate([src[:NE], pad0]),
                      jnp.concatenate([src[NE:], pad0])])
    dsts = jnp.stack([jnp.concatenate([dst[:NE], padt]),
                      jnp.concatenate([dst[NE:], padt])])
    ets = jnp.stack([jnp.concatenate([et[:NE], pad0]),
                     jnp.concatenate([et[NE:], pad0])])
    rec = jnp.stack([s.reshape(2, 16, NCHUNK, CH) for s in (srcs, dsts, ets)],
                    axis=3).reshape(2, 16 * NCHUNK * REC)
    zeros_nt = jnp.zeros((2, NPR, 16), f32)
    iota_npr = jnp.arange(NPR, dtype=i32)
    zrow = jnp.zeros((CH, D), f32)

    dinv = _k0(dsts, zeros_nt, iota_npr)
    dflat_full = dinv.reshape(2, NPR * 16)
    dflat = dflat_full[:, :N]
    di = dflat[0].reshape(N, 1)
    do = dflat[1].reshape(N, 1)

    rel_all = jnp.concatenate(
        [params['rel_graph_emb'], params['conv1']['loop_rel']], axis=0)
    h = x
    hsi, hso = _p0(x, di, do)
    names = ['conv1', 'conv2', 'conv3']
    for li, name in enumerate(names):
        p = params[name]
        rel_lo = rel_all[:R]
        a_in = _k2(hsi, rel_lo, rec[0], dflat_full[0], zrow)
        a_out = _k2(hso, rel_lo, rec[1], dflat_full[1], zrow)
        a_in = jnp.concatenate([a_in[0, :HALF], a_in[1, :HALF]], axis=0)
        a_out = jnp.concatenate([a_out[0, :HALF], a_out[1, :HALF]], axis=0)
        h, relw = _k3(a_in, a_out, h, rel_all, p, relu=(li < 2))
        if li < 2:
            hsi, hso = _p0(h, di, do)
            rel_all = jnp.concatenate(
                [relw[:R], params[names[li + 1]]['loop_rel']], axis=0)

    lwp = jnp.pad(params['lin_w'], ((0, 0), (0, 126)))
    lbp = jnp.pad(params['lin_b'].reshape(1, 2), ((0, 0), (0, 126)))
    out_full = _k4(h, batch.reshape(1, N).astype(i32),
                   rel_labels.reshape(G, 1).astype(i32),
                   params['rel_emb'], lwp, lbp)
    return out_full[:, :2]
